# baseline (speedup 1.0000x reference)
import functools

import jax
import jax.numpy as jnp
from jax.experimental import pallas as pl
from jax.experimental.pallas import tpu as pltpu
from jax.experimental.pallas import tpu_sc as plsc

_EPS = 1e-6


def _sc_sample(tl, sl):
    F = sl.shape[0]

    @functools.partial(
        pl.kernel,
        out_type=jax.ShapeDtypeStruct((4, F), jnp.float32),
        mesh=plsc.VectorSubcoreMesh(core_axis_name="c", subcore_axis_name="s"),
        scratch_types=[pltpu.VMEM((F,), jnp.float32) for _ in range(9)],
    )
    def body(tl_hbm, sl_hbm, out_hbm, t0, t1, t2, t3, sv, r0, r1, r2, r3):
        cid = jax.lax.axis_index("c")
        sid = jax.lax.axis_index("s")

        @pl.when((cid == 0) & (sid == 0))
        def _():
            pltpu.sync_copy(tl_hbm.at[0], t0)
            pltpu.sync_copy(tl_hbm.at[1], t1)
            pltpu.sync_copy(tl_hbm.at[2], t2)
            pltpu.sync_copy(tl_hbm.at[3], t3)
            pltpu.sync_copy(sl_hbm, sv)
            one = jnp.ones((16,), jnp.float32)
            zero = jnp.zeros((16,), jnp.float32)
            for g in range(F // 16):
                d = pl.ds(g * 16, 16)
                a0 = t0[d]
                a1 = t1[d]
                a2 = t2[d]
                a3 = t3[d]
                m = jnp.maximum(jnp.maximum(a0, a1), jnp.maximum(a2, a3))
                gf = jnp.where(sv[d] > 0.0, one, zero)
                w0 = jnp.where(a0 >= m, one, zero)
                w1 = jnp.where(a1 >= m, one, zero)
                w2 = jnp.where(a2 >= m, one, zero)
                n0 = one - w0
                n01 = n0 * (one - w1)
                n012 = n01 * (one - w2)
                r0[d] = gf * w0
                r1[d] = gf * n0 * w1
                r2[d] = gf * n01 * w2
                r3[d] = gf * n012
            pltpu.sync_copy(r0, out_hbm.at[0])
            pltpu.sync_copy(r1, out_hbm.at[1])
            pltpu.sync_copy(r2, out_hbm.at[2])
            pltpu.sync_copy(r3, out_hbm.at[3])

    return body(tl, sl)


def _signed_log1p(x):
    xi = jax.lax.bitcast_convert_type(x, jnp.uint32)
    sbit = xi & jnp.uint32(0x80000000)
    ax = jax.lax.bitcast_convert_type(xi & jnp.uint32(0x7FFFFFFF), jnp.float32)
    lg = jnp.log(1.0 + ax)
    li = jax.lax.bitcast_convert_type(lg, jnp.uint32)
    return jax.lax.bitcast_convert_type(li | sbit, jnp.float32)


def _body(x_ref, sel_ref, o_ref, acc_ref, xs_ref, *, nb, total_rows):
    p = pl.program_id(0)
    i = pl.program_id(1)
    rb = x_ref.shape[0]

    @pl.when(p == 0)
    def _stats_phase():
        x = x_ref[...]
        xs_ref[pl.ds(i * rb, rb), :] = x
        s = jnp.sum(x, axis=0, keepdims=True)
        ss = jnp.sum(x * x, axis=0, keepdims=True)
        mn = jnp.min(x, axis=0, keepdims=True)
        mx = jnp.max(x, axis=0, keepdims=True)

        @pl.when(i == 0)
        def _init():
            acc_ref[0:1, :] = s
            acc_ref[1:2, :] = ss
            acc_ref[2:3, :] = mn
            acc_ref[3:4, :] = mx

        @pl.when(i > 0)
        def _accum():
            acc_ref[0:1, :] += s
            acc_ref[1:2, :] += ss
            acc_ref[2:3, :] = jnp.minimum(acc_ref[2:3, :], mn)
            acc_ref[3:4, :] = jnp.maximum(acc_ref[3:4, :], mx)

        @pl.when(i == nb - 1)
        def _finalize():
            tot = acc_ref[0:1, :]
            totsq = acc_ref[1:2, :]
            cmn = acc_ref[2:3, :]
            cmx = acc_ref[3:4, :]
            mean = tot / total_rows
            var = jnp.maximum(totsq / total_rows - mean * mean, 0.0)
            std = jnp.sqrt(var)
            ma = jnp.maximum(jnp.abs(cmn), jnp.abs(cmx))
            a0 = 1.0 / (std + _EPS)
            b0 = -mean * a0
            a1 = 1.0 / (cmx - cmn + _EPS)
            b1 = -cmn * a1
            a2 = 1.0 / (ma + _EPS)
            s0 = sel_ref[0:1, :]
            s1 = sel_ref[1:2, :]
            s2 = sel_ref[2:3, :]
            ul = sel_ref[3:4, :]
            acc_ref[4:5, :] = (
                1.0 + s0 * (a0 - 1.0) + s1 * (a1 - 1.0) + s2 * (a2 - 1.0) - ul
            )
            acc_ref[5:6, :] = s0 * b0 + s1 * b1
            acc_ref[6:7, :] = ul

    @pl.when(p == 1)
    def _apply_phase():
        a = acc_ref[4:5, :]
        b = acc_ref[5:6, :]
        use_log = acc_ref[6:7, :] > 0.5
        x = xs_ref[pl.ds(i * rb, rb), :]
        lin = x * a + b
        o_ref[...] = jnp.where(use_log, _signed_log1p(x), lin)


def kernel(X, step_prob_logits, tf_prob_logits, is_train, max_only):
    B, F = X.shape
    sl = step_prob_logits.reshape(F)
    tl = tf_prob_logits.T
    sel = _sc_sample(tl, sl)
    nb = 2
    rb = B // nb

    return pl.pallas_call(
        functools.partial(_body, nb=nb, total_rows=B),
        grid=(2, nb),
        in_specs=[
            pl.BlockSpec((rb, F), lambda p, i: (i * (1 - p) + (nb - 1) * p, 0)),
            pl.BlockSpec((4, F), lambda p, i: (0, 0)),
        ],
        out_specs=pl.BlockSpec((rb, F), lambda p, i: (p * i, 0)),
        out_shape=jax.ShapeDtypeStruct((B, F), X.dtype),
        scratch_shapes=[
            pltpu.VMEM((8, F), jnp.float32),
            pltpu.VMEM((B, F), jnp.float32),
        ],
    )(X, sel)


# speedup vs baseline: 2.9272x; 2.9272x over previous
import functools

import jax
import jax.numpy as jnp
from jax.experimental import pallas as pl
from jax.experimental.pallas import tpu as pltpu

_EPS = 1e-6


def _signed_log1p(x):
    xi = jax.lax.bitcast_convert_type(x, jnp.uint32)
    sbit = xi & jnp.uint32(0x80000000)
    ax = jax.lax.bitcast_convert_type(xi & jnp.uint32(0x7FFFFFFF), jnp.float32)
    lg = jnp.log(1.0 + ax)
    li = jax.lax.bitcast_convert_type(lg, jnp.uint32)
    return jax.lax.bitcast_convert_type(li | sbit, jnp.float32)


def _body(x_ref, lg_ref, o_ref, acc_ref, xs_ref, *, nb, total_rows):
    p = pl.program_id(0)
    i = pl.program_id(1)
    rb = x_ref.shape[0]

    @pl.when(p == 0)
    def _stats_phase():
        x = x_ref[...]
        xs_ref[pl.ds(i * rb, rb), :] = x
        s = jnp.sum(x, axis=0, keepdims=True)
        ss = jnp.sum(x * x, axis=0, keepdims=True)
        mn = jnp.min(x, axis=0, keepdims=True)
        mx = jnp.max(x, axis=0, keepdims=True)

        @pl.when(i == 0)
        def _init():
            acc_ref[0:1, :] = s
            acc_ref[1:2, :] = ss
            acc_ref[2:3, :] = mn
            acc_ref[3:4, :] = mx

        @pl.when(i > 0)
        def _accum():
            acc_ref[0:1, :] += s
            acc_ref[1:2, :] += ss
            acc_ref[2:3, :] = jnp.minimum(acc_ref[2:3, :], mn)
            acc_ref[3:4, :] = jnp.maximum(acc_ref[3:4, :], mx)

        @pl.when(i == nb - 1)
        def _finalize():
            tot = acc_ref[0:1, :]
            totsq = acc_ref[1:2, :]
            cmn = acc_ref[2:3, :]
            cmx = acc_ref[3:4, :]
            mean = tot / total_rows
            var = jnp.maximum(totsq / total_rows - mean * mean, 0.0)
            std = jnp.sqrt(var)
            ma = jnp.maximum(jnp.abs(cmn), jnp.abs(cmx))
            a0 = 1.0 / (std + _EPS)
            b0 = -mean * a0
            a1 = 1.0 / (cmx - cmn + _EPS)
            b1 = -cmn * a1
            a2 = 1.0 / (ma + _EPS)
            tl = lg_ref[0:4, :]
            kmax = jnp.max(tl, axis=0, keepdims=True)
            kcap = tl.shape[0]
            jidx = jax.lax.broadcasted_iota(jnp.int32, tl.shape, 0)
            kidx = jnp.min(jnp.where(tl == kmax, jidx, kcap), axis=0, keepdims=True)
            gate = lg_ref[4:5, :] > 0.0
            use_log = gate & (kidx == 3)
            affine = gate & (kidx != 3)
            a_sel = jnp.where(kidx == 0, a0, jnp.where(kidx == 1, a1, a2))
            b_sel = jnp.where(kidx == 0, b0, jnp.where(kidx == 1, b1, 0.0))
            acc_ref[4:5, :] = jnp.where(affine, a_sel, 1.0)
            acc_ref[5:6, :] = jnp.where(affine, b_sel, 0.0)
            acc_ref[6:7, :] = jnp.where(use_log, 1.0, 0.0)

    @pl.when(p == 1)
    def _apply_phase():
        a = acc_ref[4:5, :]
        b = acc_ref[5:6, :]
        use_log = acc_ref[6:7, :] > 0.5
        x = xs_ref[pl.ds(i * rb, rb), :]
        lin = x * a + b
        o_ref[...] = jnp.where(use_log, _signed_log1p(x), lin)


def kernel(X, step_prob_logits, tf_prob_logits, is_train, max_only):
    B, F = X.shape
    K = tf_prob_logits.shape[1]
    lg = jnp.concatenate(
        [tf_prob_logits.T, step_prob_logits.reshape(1, F)], axis=0
    )
    nb = 2
    rb = B // nb

    return pl.pallas_call(
        functools.partial(_body, nb=nb, total_rows=B),
        grid=(2, nb),
        in_specs=[
            pl.BlockSpec((rb, F), lambda p, i: (i * (1 - p) + (nb - 1) * p, 0)),
            pl.BlockSpec((K + 1, F), lambda p, i: (0, 0)),
        ],
        out_specs=pl.BlockSpec((rb, F), lambda p, i: (p * i, 0)),
        out_shape=jax.ShapeDtypeStruct((B, F), X.dtype),
        scratch_shapes=[
            pltpu.VMEM((8, F), jnp.float32),
            pltpu.VMEM((B, F), jnp.float32),
        ],
    )(X, lg)
